# Initial kernel scaffold; baseline (speedup 1.0000x reference)
import jax
import jax.numpy as jnp
from jax.experimental import pallas as pl


def kernel(node_features, edge_index, edge_features, global_features, batch, edge_batch, ew1, eb1, ew2, eb2, nw1, nb1, nw2, nb2, gw1, gb1, gw2, gb2):
    raise NotImplementedError("write your pallas kernel here")


# speedup vs baseline: 3.6160x; 3.6160x over previous
import functools

import jax
import jax.numpy as jnp
from jax import lax
from jax.experimental import pallas as pl
from jax.experimental.pallas import tpu as pltpu
from jax.experimental.pallas import tpu_sc as plsc

N = 10000
E = 320000
B = 64
ND = 128
ED = 128
GD = 64
H = 256

NC = 2
NS = 16
NW = NC * NS
CH = 128
NCHUNK = E // CH
GATHER_ITERS = -(-NCHUNK // NW)
SC_CHUNKS = NCHUNK // NC
SCATTER_ITERS = -(-SC_CHUNKS // NS)
NP = 10240
NSTRIPE = NP // NS

BLK_E = 1280
GRID_E = E // BLK_E
BLK_N = 1000
GRID_N = N // BLK_N

def _softplus(x):
    return jnp.maximum(x, 0.0) + jnp.log1p(jnp.exp(-jnp.abs(x)))


def _mesh():
    return plsc.VectorSubcoreMesh(core_axis_name="c", subcore_axis_name="s",
                                  num_cores=NC, num_subcores=NS)


def _sc_gather_body(nf_hbm, row_hbm, col_hbm, outa_hbm, outb_hbm,
                    idxa, idxb, bufa, bufb, sem):
    c = lax.axis_index("c")
    s = lax.axis_index("s")
    wid = s * NC + c

    def body(i, carry):
        cid = i * NW + wid

        @pl.when(cid < NCHUNK)
        def _():
            base = cid * CH
            pltpu.sync_copy(row_hbm.at[pl.ds(base, CH)], idxa)
            pltpu.sync_copy(col_hbm.at[pl.ds(base, CH)], idxb)
            cpa = pltpu.async_copy(nf_hbm.at[idxa], bufa, sem)
            cpb = pltpu.async_copy(nf_hbm.at[idxb], bufb, sem)
            cpa.wait()
            cpb.wait()
            pltpu.sync_copy(bufa, outa_hbm.at[pl.ds(base, CH)])
            pltpu.sync_copy(bufb, outb_hbm.at[pl.ds(base, CH)])

        return carry

    lax.fori_loop(0, GATHER_ITERS, body, 0)


def _sc_gather(nf, row, col):
    return pl.kernel(
        _sc_gather_body,
        out_type=(jax.ShapeDtypeStruct((E, ND), jnp.float32),
                  jax.ShapeDtypeStruct((E, ND), jnp.float32)),
        mesh=_mesh(),
        scratch_types=[
            pltpu.VMEM((CH,), jnp.int32),
            pltpu.VMEM((CH,), jnp.int32),
            pltpu.VMEM((CH, ND), jnp.float32),
            pltpu.VMEM((CH, ND), jnp.float32),
            pltpu.SemaphoreType.DMA,
        ],
    )(nf, row, col)


def _sc_scatter_body(ef_hbm, row_hbm, zeros_hbm, out_hbm, idx, buf, acc):
    c = lax.axis_index("c")
    s = lax.axis_index("s")
    pltpu.sync_copy(zeros_hbm, acc.at[pl.ds(s * NSTRIPE, NSTRIPE)])
    plsc.subcore_barrier()

    def body(i, carry):
        lcid = i * NS + s

        @pl.when(lcid < SC_CHUNKS)
        def _():
            base = (c * SC_CHUNKS + lcid) * CH
            pltpu.sync_copy(row_hbm.at[pl.ds(base, CH)], idx)
            pltpu.sync_copy(ef_hbm.at[pl.ds(base, CH)], buf)
            pltpu.sync_copy(buf, acc.at[idx], add=True)

        return carry

    lax.fori_loop(0, SCATTER_ITERS, body, 0)
    plsc.subcore_barrier()
    pltpu.sync_copy(acc.at[pl.ds(s * NSTRIPE, NSTRIPE)],
                    out_hbm.at[c, pl.ds(s * NSTRIPE, NSTRIPE)])


def _sc_scatter(ef_new, row, zeros_stripe):
    return pl.kernel(
        _sc_scatter_body,
        out_type=jax.ShapeDtypeStruct((NC, NP, ND), jnp.float32),
        mesh=_mesh(),
        scratch_types=[
            pltpu.VMEM((CH,), jnp.int32),
            pltpu.VMEM((CH, ED), jnp.float32),
            pltpu.VMEM_SHARED((NP, ND), jnp.float32),
        ],
    )(ef_new, row, zeros_stripe)


def _prep_body(gf_ref, ew1g_ref, eb1_ref, nw1g_ref, nb1_ref, ge_ref, gn_ref):
    gf = gf_ref[...]
    ge_ref[...] = jnp.dot(gf, ew1g_ref[...],
                          preferred_element_type=jnp.float32) + eb1_ref[...]
    gn_ref[...] = jnp.dot(gf, nw1g_ref[...],
                          preferred_element_type=jnp.float32) + nb1_ref[...]


def _edge_body(eb_ref, a_ref, b_ref, e_ref, wr_ref, wc_ref, we_ref, ge_ref,
               w2_ref, b2_ref, out_ref, se_ref, ce_ref):
    i = pl.program_id(0)
    e = e_ref[...]
    pre = jnp.dot(a_ref[...], wr_ref[...], preferred_element_type=jnp.float32)
    pre = pre + jnp.dot(b_ref[...], wc_ref[...],
                        preferred_element_type=jnp.float32)
    pre = pre + jnp.dot(e, we_ref[...], preferred_element_type=jnp.float32)
    eb = eb_ref[0]
    onehot = (eb == lax.broadcasted_iota(jnp.int32, (1, B), 1)
              ).astype(jnp.float32)
    pre = pre + jnp.dot(onehot, ge_ref[...],
                        preferred_element_type=jnp.float32)
    h = _softplus(pre)
    out = e + jnp.dot(h, w2_ref[...],
                      preferred_element_type=jnp.float32) + b2_ref[...]
    out_ref[...] = out
    pooled = lax.dot_general(onehot, out, (((0,), (0,)), ((), ())),
                             preferred_element_type=jnp.float32)
    cnt = jnp.sum(onehot, axis=0)

    @pl.when(i == 0)
    def _():
        se_ref[...] = jnp.zeros_like(se_ref)
        ce_ref[...] = jnp.zeros_like(ce_ref)

    se_ref[...] += pooled
    ce_ref[...] += jnp.broadcast_to(cnt[:, None], (B, ED))


def _node_body(bt_ref, nf_ref, m_ref, w1n_ref, w1m_ref, gn_ref, w2_ref,
               b2_ref, out_ref, sn_ref, cn_ref):
    i = pl.program_id(0)
    nf = nf_ref[...]
    m = m_ref[0] + m_ref[1]
    pre = jnp.dot(nf, w1n_ref[...], preferred_element_type=jnp.float32)
    pre = pre + jnp.dot(m, w1m_ref[...], preferred_element_type=jnp.float32)
    bt = bt_ref[0]
    onehot = (bt == lax.broadcasted_iota(jnp.int32, (1, B), 1)
              ).astype(jnp.float32)
    pre = pre + jnp.dot(onehot, gn_ref[...],
                        preferred_element_type=jnp.float32)
    h = _softplus(pre)
    out = nf + jnp.dot(h, w2_ref[...],
                       preferred_element_type=jnp.float32) + b2_ref[...]
    out_ref[...] = out
    pooled = lax.dot_general(onehot, out, (((0,), (0,)), ((), ())),
                             preferred_element_type=jnp.float32)
    cnt = jnp.sum(onehot, axis=0)

    @pl.when(i == 0)
    def _():
        sn_ref[...] = jnp.zeros_like(sn_ref)
        cn_ref[...] = jnp.zeros_like(cn_ref)

    sn_ref[...] += pooled
    cn_ref[...] += jnp.broadcast_to(cnt[:, None], (B, ND))


def _global_body(sn_ref, cn_ref, se_ref, ce_ref, gf_ref, g1n_ref, g1e_ref,
                 g1g_ref, gb1_ref, gw2_ref, gb2_ref, out_ref):
    gn = sn_ref[...] / cn_ref[...]
    ge = se_ref[...] / ce_ref[...]
    gf = gf_ref[...]
    pre = jnp.dot(gn, g1n_ref[...], preferred_element_type=jnp.float32)
    pre = pre + jnp.dot(ge, g1e_ref[...], preferred_element_type=jnp.float32)
    pre = pre + jnp.dot(gf, g1g_ref[...], preferred_element_type=jnp.float32)
    pre = pre + gb1_ref[...]
    h = _softplus(pre)
    out_ref[...] = gf + jnp.dot(h, gw2_ref[...],
                                preferred_element_type=jnp.float32) + gb2_ref[...]


def kernel(node_features, edge_index, edge_features, global_features, batch,
           edge_batch, ew1, eb1, ew2, eb2, nw1, nb1, nw2, nb2, gw1, gb1,
           gw2, gb2):
    f32 = jnp.float32
    row = edge_index[0]
    col = edge_index[1]
    wr, wc, we, ew1g = ew1[:ND], ew1[ND:2 * ND], ew1[2 * ND:2 * ND + ED], \
        ew1[2 * ND + ED:]
    nw1n, nw1m, nw1g = nw1[:ND], nw1[ND:ND + ED], nw1[ND + ED:]
    g1n, g1e, g1g = gw1[:ND], gw1[ND:ND + ED], gw1[ND + ED:]

    ge_tab, gn_tab = pl.pallas_call(
        _prep_body,
        out_shape=(jax.ShapeDtypeStruct((B, H), f32),
                   jax.ShapeDtypeStruct((B, H), f32)),
    )(global_features, ew1g, eb1.reshape(1, H), nw1g, nb1.reshape(1, H))

    a_gath, b_gath = _sc_gather(node_features, row, col)

    eb3 = edge_batch.reshape(GRID_E, BLK_E, 1)
    ef_new, se_sum, ce_cnt = pl.pallas_call(
        _edge_body,
        grid=(GRID_E,),
        in_specs=[
            pl.BlockSpec((1, BLK_E, 1), lambda i: (i, 0, 0)),
            pl.BlockSpec((BLK_E, ND), lambda i: (i, 0)),
            pl.BlockSpec((BLK_E, ND), lambda i: (i, 0)),
            pl.BlockSpec((BLK_E, ED), lambda i: (i, 0)),
            pl.BlockSpec((ND, H), lambda i: (0, 0)),
            pl.BlockSpec((ND, H), lambda i: (0, 0)),
            pl.BlockSpec((ED, H), lambda i: (0, 0)),
            pl.BlockSpec((B, H), lambda i: (0, 0)),
            pl.BlockSpec((H, ED), lambda i: (0, 0)),
            pl.BlockSpec((1, ED), lambda i: (0, 0)),
        ],
        out_specs=[
            pl.BlockSpec((BLK_E, ED), lambda i: (i, 0)),
            pl.BlockSpec((B, ED), lambda i: (0, 0)),
            pl.BlockSpec((B, ED), lambda i: (0, 0)),
        ],
        out_shape=(jax.ShapeDtypeStruct((E, ED), f32),
                   jax.ShapeDtypeStruct((B, ED), f32),
                   jax.ShapeDtypeStruct((B, ED), f32)),
    )(eb3, a_gath, b_gath, edge_features, wr, wc, we, ge_tab, ew2,
      eb2.reshape(1, ED))

    msgs2 = _sc_scatter(ef_new, row, jnp.zeros((NSTRIPE, ND), f32))

    bt3 = batch.reshape(GRID_N, BLK_N, 1)
    nf_new, sn_sum, cn_cnt = pl.pallas_call(
        _node_body,
        grid=(GRID_N,),
        in_specs=[
            pl.BlockSpec((1, BLK_N, 1), lambda i: (i, 0, 0)),
            pl.BlockSpec((BLK_N, ND), lambda i: (i, 0)),
            pl.BlockSpec((NC, BLK_N, ND), lambda i: (0, i, 0)),
            pl.BlockSpec((ND, H), lambda i: (0, 0)),
            pl.BlockSpec((ED, H), lambda i: (0, 0)),
            pl.BlockSpec((B, H), lambda i: (0, 0)),
            pl.BlockSpec((H, ND), lambda i: (0, 0)),
            pl.BlockSpec((1, ND), lambda i: (0, 0)),
        ],
        out_specs=[
            pl.BlockSpec((BLK_N, ND), lambda i: (i, 0)),
            pl.BlockSpec((B, ND), lambda i: (0, 0)),
            pl.BlockSpec((B, ND), lambda i: (0, 0)),
        ],
        out_shape=(jax.ShapeDtypeStruct((N, ND), f32),
                   jax.ShapeDtypeStruct((B, ND), f32),
                   jax.ShapeDtypeStruct((B, ND), f32)),
    )(bt3, node_features, msgs2, nw1n, nw1m, gn_tab, nw2,
      nb2.reshape(1, ND))

    gf_new = pl.pallas_call(
        _global_body,
        out_shape=jax.ShapeDtypeStruct((B, GD), f32),
    )(sn_sum, cn_cnt, se_sum, ce_cnt, global_features, g1n, g1e, g1g,
      gb1.reshape(1, H), gw2, gb2.reshape(1, GD))

    return (nf_new, ef_new, gf_new)


# speedup vs baseline: 3.7116x; 1.0264x over previous
import functools

import jax
import jax.numpy as jnp
from jax import lax
from jax.experimental import pallas as pl
from jax.experimental.pallas import tpu as pltpu
from jax.experimental.pallas import tpu_sc as plsc

N = 10000
E = 320000
B = 64
ND = 128
ED = 128
GD = 64
H = 256
HH = H // 2

NC = 2
NS = 16
NW = NC * NS
CH = 128
NCHUNK = E // CH
GATHER_ITERS = -(-NCHUNK // NW)
SC_CHUNKS = NCHUNK // NC
SCATTER_ITERS = -(-SC_CHUNKS // NS)
NP = 10240
NSTRIPE = NP // NS

BLK_E = 1280
GRID_E = E // BLK_E
BLK_N = 1000
GRID_N = N // BLK_N


def _softplus(x):
    return jnp.maximum(x, 0.0) + jnp.log1p(jnp.exp(-jnp.abs(x)))


def _mesh():
    return plsc.VectorSubcoreMesh(core_axis_name="c", subcore_axis_name="s",
                                  num_cores=NC, num_subcores=NS)


def _pack_halves(x):
    lo = lax.bitcast_convert_type(x[:, :HH].astype(jnp.bfloat16), jnp.int16)
    hi = lax.bitcast_convert_type(x[:, HH:].astype(jnp.bfloat16), jnp.int16)
    lo32 = lax.bitwise_and(lo.astype(jnp.int32), jnp.int32(0xFFFF))
    return lax.bitwise_or(lo32, lax.shift_left(hi.astype(jnp.int32), 16))


def _unpack_halves(p):
    lo = lax.bitcast_convert_type(lax.shift_left(p, 16), jnp.float32)
    hi = lax.bitcast_convert_type(
        lax.bitwise_and(p, jnp.int32(-65536)), jnp.float32)
    return lo, hi


def _sc_gather_body(pa_hbm, pb_hbm, row_hbm, col_hbm, outa_hbm, outb_hbm,
                    idxa, idxb, bufa, bufb, sem):
    c = lax.axis_index("c")
    s = lax.axis_index("s")
    wid = s * NC + c

    def body(i, carry):
        cid = i * NW + wid

        @pl.when(cid < NCHUNK)
        def _():
            base = cid * CH
            pltpu.sync_copy(row_hbm.at[pl.ds(base, CH)], idxa)
            pltpu.sync_copy(col_hbm.at[pl.ds(base, CH)], idxb)
            cpa = pltpu.async_copy(pa_hbm.at[idxa], bufa, sem)
            cpb = pltpu.async_copy(pb_hbm.at[idxb], bufb, sem)
            cpa.wait()
            cpb.wait()
            pltpu.sync_copy(bufa, outa_hbm.at[pl.ds(base, CH)])
            pltpu.sync_copy(bufb, outb_hbm.at[pl.ds(base, CH)])

        return carry

    lax.fori_loop(0, GATHER_ITERS, body, 0)


def _sc_gather(pa_pack, pb_pack, row, col):
    return pl.kernel(
        _sc_gather_body,
        out_type=(jax.ShapeDtypeStruct((E, HH), jnp.int32),
                  jax.ShapeDtypeStruct((E, HH), jnp.int32)),
        mesh=_mesh(),
        scratch_types=[
            pltpu.VMEM((CH,), jnp.int32),
            pltpu.VMEM((CH,), jnp.int32),
            pltpu.VMEM((CH, HH), jnp.int32),
            pltpu.VMEM((CH, HH), jnp.int32),
            pltpu.SemaphoreType.DMA,
        ],
    )(pa_pack, pb_pack, row, col)


def _sc_scatter_body(ef_hbm, row_hbm, zeros_hbm, out_hbm, idx, buf, acc):
    c = lax.axis_index("c")
    s = lax.axis_index("s")
    pltpu.sync_copy(zeros_hbm, acc.at[pl.ds(s * NSTRIPE, NSTRIPE)])
    plsc.subcore_barrier()

    def body(i, carry):
        lcid = i * NS + s

        @pl.when(lcid < SC_CHUNKS)
        def _():
            base = (c * SC_CHUNKS + lcid) * CH
            pltpu.sync_copy(row_hbm.at[pl.ds(base, CH)], idx)
            pltpu.sync_copy(ef_hbm.at[pl.ds(base, CH)], buf)
            pltpu.sync_copy(buf, acc.at[idx], add=True)

        return carry

    lax.fori_loop(0, SCATTER_ITERS, body, 0)
    plsc.subcore_barrier()
    pltpu.sync_copy(acc.at[pl.ds(s * NSTRIPE, NSTRIPE)],
                    out_hbm.at[c, pl.ds(s * NSTRIPE, NSTRIPE)])


def _sc_scatter(ef_new, row, zeros_stripe):
    return pl.kernel(
        _sc_scatter_body,
        out_type=jax.ShapeDtypeStruct((NC, NP, ND), jnp.float32),
        mesh=_mesh(),
        scratch_types=[
            pltpu.VMEM((CH,), jnp.int32),
            pltpu.VMEM((CH, ED), jnp.float32),
            pltpu.VMEM_SHARED((NP, ND), jnp.float32),
        ],
    )(ef_new, row, zeros_stripe)


def _prep_body(gf_ref, ew1g_ref, eb1_ref, nw1g_ref, nb1_ref, ge_ref, gn_ref):
    gf = gf_ref[...]
    ge_ref[...] = jnp.dot(gf, ew1g_ref[...],
                          preferred_element_type=jnp.float32) + eb1_ref[...]
    gn_ref[...] = jnp.dot(gf, nw1g_ref[...],
                          preferred_element_type=jnp.float32) + nb1_ref[...]


def _proj_body(nf_ref, wr_ref, wc_ref, pa_ref, pb_ref):
    nf = nf_ref[...].astype(jnp.bfloat16)
    pa = jnp.dot(nf, wr_ref[...], preferred_element_type=jnp.float32)
    pb = jnp.dot(nf, wc_ref[...], preferred_element_type=jnp.float32)
    pa_ref[...] = _pack_halves(pa)
    pb_ref[...] = _pack_halves(pb)


def _edge_body(eb_ref, a_ref, b_ref, e_ref, welo_ref, wehi_ref, gelo_ref,
               gehi_ref, w2lo_ref, w2hi_ref, b2_ref,
               out_ref, se_ref, ce_ref):
    i = pl.program_id(0)
    e = e_ref[...]
    e_bf = e.astype(jnp.bfloat16)
    a_lo, a_hi = _unpack_halves(a_ref[...])
    b_lo, b_hi = _unpack_halves(b_ref[...])
    eb = eb_ref[0]
    onehot = (eb == lax.broadcasted_iota(jnp.int32, (1, B), 1)
              ).astype(jnp.float32)
    oh_bf = onehot.astype(jnp.bfloat16)
    pre_lo = a_lo + b_lo + \
        jnp.dot(e_bf, welo_ref[...], preferred_element_type=jnp.float32) + \
        jnp.dot(oh_bf, gelo_ref[...], preferred_element_type=jnp.float32)
    pre_hi = a_hi + b_hi + \
        jnp.dot(e_bf, wehi_ref[...], preferred_element_type=jnp.float32) + \
        jnp.dot(oh_bf, gehi_ref[...], preferred_element_type=jnp.float32)
    h_lo = _softplus(pre_lo).astype(jnp.bfloat16)
    h_hi = _softplus(pre_hi).astype(jnp.bfloat16)
    out = e + b2_ref[...] + \
        jnp.dot(h_lo, w2lo_ref[...], preferred_element_type=jnp.float32) + \
        jnp.dot(h_hi, w2hi_ref[...], preferred_element_type=jnp.float32)
    out_ref[...] = out
    pooled = lax.dot_general(onehot, out, (((0,), (0,)), ((), ())),
                             preferred_element_type=jnp.float32)
    cnt = jnp.sum(onehot, axis=0)

    @pl.when(i == 0)
    def _():
        se_ref[...] = jnp.zeros_like(se_ref)
        ce_ref[...] = jnp.zeros_like(ce_ref)

    se_ref[...] += pooled
    ce_ref[...] += jnp.broadcast_to(cnt[:, None], (B, ED))


def _node_body(bt_ref, nf_ref, m_ref, w1n_ref, w1m_ref, gn_ref, w2_ref,
               b2_ref, out_ref, sn_ref, cn_ref):
    i = pl.program_id(0)
    nf = nf_ref[...]
    m = m_ref[0] + m_ref[1]
    pre = jnp.dot(nf, w1n_ref[...], preferred_element_type=jnp.float32)
    pre = pre + jnp.dot(m, w1m_ref[...], preferred_element_type=jnp.float32)
    bt = bt_ref[0]
    onehot = (bt == lax.broadcasted_iota(jnp.int32, (1, B), 1)
              ).astype(jnp.float32)
    pre = pre + jnp.dot(onehot, gn_ref[...],
                        preferred_element_type=jnp.float32)
    h = _softplus(pre)
    out = nf + jnp.dot(h, w2_ref[...],
                       preferred_element_type=jnp.float32) + b2_ref[...]
    out_ref[...] = out
    pooled = lax.dot_general(onehot, out, (((0,), (0,)), ((), ())),
                             preferred_element_type=jnp.float32)
    cnt = jnp.sum(onehot, axis=0)

    @pl.when(i == 0)
    def _():
        sn_ref[...] = jnp.zeros_like(sn_ref)
        cn_ref[...] = jnp.zeros_like(cn_ref)

    sn_ref[...] += pooled
    cn_ref[...] += jnp.broadcast_to(cnt[:, None], (B, ND))


def _global_body(sn_ref, cn_ref, se_ref, ce_ref, gf_ref, g1n_ref, g1e_ref,
                 g1g_ref, gb1_ref, gw2_ref, gb2_ref, out_ref):
    gn = sn_ref[...] / cn_ref[...]
    ge = se_ref[...] / ce_ref[...]
    gf = gf_ref[...]
    pre = jnp.dot(gn, g1n_ref[...], preferred_element_type=jnp.float32)
    pre = pre + jnp.dot(ge, g1e_ref[...], preferred_element_type=jnp.float32)
    pre = pre + jnp.dot(gf, g1g_ref[...], preferred_element_type=jnp.float32)
    pre = pre + gb1_ref[...]
    h = _softplus(pre)
    out_ref[...] = gf + jnp.dot(h, gw2_ref[...],
                                preferred_element_type=jnp.float32) + gb2_ref[...]


def kernel(node_features, edge_index, edge_features, global_features, batch,
           edge_batch, ew1, eb1, ew2, eb2, nw1, nb1, nw2, nb2, gw1, gb1,
           gw2, gb2):
    f32 = jnp.float32
    bf16 = jnp.bfloat16
    row = edge_index[0]
    col = edge_index[1]
    wr, wc, we, ew1g = ew1[:ND], ew1[ND:2 * ND], ew1[2 * ND:2 * ND + ED], \
        ew1[2 * ND + ED:]
    nw1n, nw1m, nw1g = nw1[:ND], nw1[ND:ND + ED], nw1[ND + ED:]
    g1n, g1e, g1g = gw1[:ND], gw1[ND:ND + ED], gw1[ND + ED:]

    ge_tab, gn_tab = pl.pallas_call(
        _prep_body,
        out_shape=(jax.ShapeDtypeStruct((B, H), f32),
                   jax.ShapeDtypeStruct((B, H), f32)),
    )(global_features, ew1g, eb1.reshape(1, H), nw1g, nb1.reshape(1, H))

    pa_pack, pb_pack = pl.pallas_call(
        _proj_body,
        grid=(GRID_N,),
        in_specs=[
            pl.BlockSpec((BLK_N, ND), lambda i: (i, 0)),
            pl.BlockSpec((ND, H), lambda i: (0, 0)),
            pl.BlockSpec((ND, H), lambda i: (0, 0)),
        ],
        out_specs=[
            pl.BlockSpec((BLK_N, HH), lambda i: (i, 0)),
            pl.BlockSpec((BLK_N, HH), lambda i: (i, 0)),
        ],
        out_shape=(jax.ShapeDtypeStruct((N, HH), jnp.int32),
                   jax.ShapeDtypeStruct((N, HH), jnp.int32)),
    )(node_features, wr.astype(bf16), wc.astype(bf16))

    a_gath, b_gath = _sc_gather(pa_pack, pb_pack, row, col)

    eb3 = edge_batch.reshape(GRID_E, BLK_E, 1)
    ef_new, se_sum, ce_cnt = pl.pallas_call(
        _edge_body,
        grid=(GRID_E,),
        in_specs=[
            pl.BlockSpec((1, BLK_E, 1), lambda i: (i, 0, 0)),
            pl.BlockSpec((BLK_E, HH), lambda i: (i, 0)),
            pl.BlockSpec((BLK_E, HH), lambda i: (i, 0)),
            pl.BlockSpec((BLK_E, ED), lambda i: (i, 0)),
            pl.BlockSpec((ED, HH), lambda i: (0, 0)),
            pl.BlockSpec((ED, HH), lambda i: (0, 0)),
            pl.BlockSpec((B, HH), lambda i: (0, 0)),
            pl.BlockSpec((B, HH), lambda i: (0, 0)),
            pl.BlockSpec((HH, ED), lambda i: (0, 0)),
            pl.BlockSpec((HH, ED), lambda i: (0, 0)),
            pl.BlockSpec((1, ED), lambda i: (0, 0)),
        ],
        out_specs=[
            pl.BlockSpec((BLK_E, ED), lambda i: (i, 0)),
            pl.BlockSpec((B, ED), lambda i: (0, 0)),
            pl.BlockSpec((B, ED), lambda i: (0, 0)),
        ],
        out_shape=(jax.ShapeDtypeStruct((E, ED), f32),
                   jax.ShapeDtypeStruct((B, ED), f32),
                   jax.ShapeDtypeStruct((B, ED), f32)),
    )(eb3, a_gath, b_gath, edge_features,
      we[:, :HH].astype(bf16), we[:, HH:].astype(bf16),
      ge_tab[:, :HH].astype(bf16), ge_tab[:, HH:].astype(bf16),
      ew2[:HH].astype(bf16), ew2[HH:].astype(bf16),
      eb2.reshape(1, ED))

    msgs2 = _sc_scatter(ef_new, row, jnp.zeros((NSTRIPE, ND), f32))

    bt3 = batch.reshape(GRID_N, BLK_N, 1)
    nf_new, sn_sum, cn_cnt = pl.pallas_call(
        _node_body,
        grid=(GRID_N,),
        in_specs=[
            pl.BlockSpec((1, BLK_N, 1), lambda i: (i, 0, 0)),
            pl.BlockSpec((BLK_N, ND), lambda i: (i, 0)),
            pl.BlockSpec((NC, BLK_N, ND), lambda i: (0, i, 0)),
            pl.BlockSpec((ND, H), lambda i: (0, 0)),
            pl.BlockSpec((ED, H), lambda i: (0, 0)),
            pl.BlockSpec((B, H), lambda i: (0, 0)),
            pl.BlockSpec((H, ND), lambda i: (0, 0)),
            pl.BlockSpec((1, ND), lambda i: (0, 0)),
        ],
        out_specs=[
            pl.BlockSpec((BLK_N, ND), lambda i: (i, 0)),
            pl.BlockSpec((B, ND), lambda i: (0, 0)),
            pl.BlockSpec((B, ND), lambda i: (0, 0)),
        ],
        out_shape=(jax.ShapeDtypeStruct((N, ND), f32),
                   jax.ShapeDtypeStruct((B, ND), f32),
                   jax.ShapeDtypeStruct((B, ND), f32)),
    )(bt3, node_features, msgs2, nw1n, nw1m, gn_tab, nw2,
      nb2.reshape(1, ND))

    gf_new = pl.pallas_call(
        _global_body,
        out_shape=jax.ShapeDtypeStruct((B, GD), f32),
    )(sn_sum, cn_cnt, se_sum, ce_cnt, global_features, g1n, g1e, g1g,
      gb1.reshape(1, H), gw2, gb2.reshape(1, GD))

    return (nf_new, ef_new, gf_new)
